# baseline (speedup 1.0000x reference)
import functools

import jax
import jax.numpy as jnp
from jax import lax
from jax.experimental import pallas as pl
from jax.experimental.pallas import tpu as pltpu
from jax.experimental.pallas import tpu_sc as plsc

_B, _N, _K = 4, 2048, 16
_BN = _B * _N
_RANK = 32
_OUT = 128
_NG = 32


_TOPK_ROWS = 256


def _topk_body(dist_ref, idx_ref):
    x = dist_ref[...]
    r = x.shape[0]
    colf = lax.broadcasted_iota(jnp.int32, (r, _N), 1).astype(jnp.float32)
    lane = lax.broadcasted_iota(jnp.int32, (r, _K), 1)
    acc = jnp.zeros((r, _K), jnp.int32)
    for j in range(_K):
        m = jnp.min(x, axis=1, keepdims=True)
        i = jnp.min(jnp.where(x == m, colf, jnp.float32(_N)),
                    axis=1)
        acc = jnp.where(lane == j, i[:, None].astype(jnp.int32), acc)
        x = jnp.where(colf == i[:, None], jnp.inf, x)
    base = pl.program_id(0) * _TOPK_ROWS // _N * _N
    idx_ref[...] = acc + base


def _topk(dist2):
    rows = dist2.shape[0]
    return pl.pallas_call(
        _topk_body,
        grid=(rows // _TOPK_ROWS,),
        in_specs=[pl.BlockSpec((_TOPK_ROWS, _N), lambda i: (i, 0))],
        out_specs=pl.BlockSpec((_TOPK_ROWS, _K), lambda i: (i, 0)),
        out_shape=jax.ShapeDtypeStruct((rows, _K), jnp.int32),
    )(dist2)


_NW = 32
_CH = 128
_GW = 128


def _sc_sqrt(x):
    xi = lax.bitcast_convert_type(x, jnp.int32)
    s = lax.bitcast_convert_type((xi >> 1) + 0x1FBD1DF6, jnp.float32)
    for _ in range(3):
        s = 0.5 * (s + x / s)
    return s


def _sc_geom(table_flat, gidx_flat):
    m = gidx_flat.shape[0]
    per_w = m // _NW
    npts = per_w // _K
    tlen = table_flat.shape[0]
    mesh = plsc.VectorSubcoreMesh(core_axis_name="c", subcore_axis_name="s")

    @functools.partial(
        pl.kernel,
        mesh=mesh,
        out_type=jax.ShapeDtypeStruct((m * 8,), jnp.float32),
        scratch_types=[
            pltpu.VMEM((tlen,), jnp.float32),
            pltpu.VMEM((per_w,), jnp.int32),
            pltpu.VMEM((per_w * 8,), jnp.float32),
        ],
        compiler_params=pltpu.CompilerParams(needs_layout_passes=False),
    )
    def k(table_hbm, idx_hbm, out_hbm, table_v, idx_v, obuf):
        wid = lax.axis_index("s") * 2 + lax.axis_index("c")
        pltpu.sync_copy(table_hbm, table_v)
        pltpu.sync_copy(idx_hbm.at[pl.ds(wid * per_w, per_w)], idx_v)
        lanes = lax.iota(jnp.int32, 16)

        def body(p, carry):
            idx16 = idx_v[pl.ds(p * _K, _K)]
            pg = wid * npts + p

            def nbr(c):
                return plsc.load_gather(table_v, [idx16 * 8 + c])

            def cen(c):
                return plsc.load_gather(
                    table_v, [jnp.full((16,), pg * 8 + c, jnp.int32)])

            px, py, pz = nbr(0), nbr(1), nbr(2)
            nx, ny, nz = nbr(3), nbr(4), nbr(5)
            cx, cy, cz = cen(0), cen(1), cen(2)
            wx, wy, wz = cen(3), cen(4), cen(5)
            mx = jnp.broadcast_to(jnp.sum(px) * (1.0 / _K), (16,))
            my = jnp.broadcast_to(jnp.sum(py) * (1.0 / _K), (16,))
            mz = jnp.broadcast_to(jnp.sum(pz) * (1.0 / _K), (16,))
            l1x, l1y, l1z = mx - px, my - py, mz - pz
            l2x, l2y, l2z = px - cx, py - cy, pz - cz
            l3x, l3y, l3z = cx - mx, cy - my, cz - mz
            l1n = _sc_sqrt(l1x * l1x + l1y * l1y + l1z * l1z)
            l2n = _sc_sqrt(l2x * l2x + l2y * l2y + l2z * l2z)
            l3n = _sc_sqrt(l3x * l3x + l3y * l3y + l3z * l3z)
            t1 = (l1x * l2x + l1y * l2y + l1z * l2z) / (l1n * l2n + 1e-7)
            t2 = (l2x * l3x + l2y * l3y + l2z * l3z) / (l2n * l3n + 1e-7)
            t3 = (l3x * l1x + l3y * l1y + l3z * l1z) / (l3n * l1n + 1e-7)
            cosv = nx * wx + ny * wy + nz * wz
            zero = jnp.zeros((16,), jnp.float32)
            obase = p * (_K * 8)
            for c, val in enumerate([l1n, l2n, l3n, t1, t2, t3, cosv, zero]):
                plsc.store_scatter(obuf, [obase + lanes * 8 + c], val)
            return carry

        lax.fori_loop(0, npts, body, 0)
        pltpu.sync_copy(obuf, out_hbm.at[pl.ds(wid * per_w * 8, per_w * 8)])

    return k(table_flat, gidx_flat)


def _sc_gather(table, gidx2, d):
    m = gidx2.shape[0] * _CH
    per_w = m // _NW
    nch = per_w // _CH
    mesh = plsc.VectorSubcoreMesh(core_axis_name="c", subcore_axis_name="s")

    @functools.partial(
        pl.kernel,
        mesh=mesh,
        out_type=jax.ShapeDtypeStruct((m, d), jnp.float32),
        scratch_types=[
            pltpu.VMEM((nch, _CH), jnp.int32),
            pltpu.VMEM((2, _CH, d), jnp.float32),
            pltpu.SemaphoreType.DMA,
        ],
    )
    def k(table_hbm, idx_hbm, out_hbm, idx_v, buf, sem):
        wid = lax.axis_index("s") * 2 + lax.axis_index("c")
        base = wid * per_w
        pltpu.sync_copy(idx_hbm.at[pl.ds(wid * nch, nch)], idx_v)
        pltpu.async_copy(table_hbm.at[idx_v.at[0]], buf.at[0], sem)

        def body(j, _):
            pltpu.make_async_copy(
                table_hbm.at[idx_v.at[0]], buf.at[0], sem).wait()

            @pl.when(j + 1 < nch)
            def _fire():
                pltpu.async_copy(
                    table_hbm.at[idx_v.at[j + 1]],
                    buf.at[lax.rem(j + 1, 2)], sem)

            pltpu.sync_copy(buf.at[lax.rem(j, 2)],
                            out_hbm.at[pl.ds(base + j * _CH, _CH)])
            return _

        lax.fori_loop(0, nch, body, 0)

    return k(table, gidx2)


def _ln_mx(x, g, b, c):
    mu = jnp.mean(x, axis=-1, keepdims=True)
    xc = x - mu
    v = jnp.mean(xc * xc, axis=-1, keepdims=True)
    return xc / jnp.sqrt(v + 1e-5) * g + b


def _mlp_kern(ri, w):
    h = jnp.dot(ri, w["kw0T"], preferred_element_type=jnp.float32) + w["kb0"]
    h = jnp.maximum(_ln_mx(h, w["kg0"], w["kbt0"], 32), 0.0)
    h = jnp.dot(h, w["kw1T"], preferred_element_type=jnp.float32) + w["kb1"]
    h = jnp.maximum(_ln_mx(h, w["kg1"], w["kbt1"], 32), 0.0)
    return jnp.dot(h, w["kw2T"], preferred_element_type=jnp.float32) + w["kb2"]


_P1 = 256


def _stage1_body(ri_ref, kw0T, kb0, kg0, kbt0, kw1T, kb1, kg1,
                 kbt1, kw2T, kb2, owTe, owTo, ob, lng, lnb, awT, ab,
                 f_ref, glob_ref, acc):
    i = pl.program_id(0)
    per_batch = _N // _P1

    @pl.when(i % per_batch == 0)
    def _():
        acc[...] = jnp.full((1, _NG), -jnp.inf, jnp.float32)

    p = _P1
    ri = ri_ref[...]
    l2n = ri[:, 1:2].reshape(p, _K, 1)
    cosv = ri[:, 6:7].reshape(p, _K, 1)
    w = dict(kw0T=kw0T[...], kb0=kb0[...], kg0=kg0[...], kbt0=kbt0[...],
             kw1T=kw1T[...], kb1=kb1[...], kg1=kg1[...], kbt1=kbt1[...],
             kw2T=kw2T[...], kb2=kb2[...])
    kern = _mlp_kern(ri, w).reshape(p, _K, _RANK)
    a0 = jnp.sum(kern * l2n, axis=1)
    a1 = jnp.sum(kern * cosv, axis=1)
    out = (jnp.dot(a0, owTe[...], preferred_element_type=jnp.float32)
           + jnp.dot(a1, owTo[...], preferred_element_type=jnp.float32)
           + ob[...])
    out = _ln_mx(out, lng[...], lnb[...], _OUT)
    f_ref[...] = out
    tran = jnp.dot(out, awT[...], preferred_element_type=jnp.float32) + ab[...]
    acc[...] = jnp.maximum(acc[...], jnp.max(tran, axis=0, keepdims=True))
    glob_ref[...] = acc[...].reshape(1, 1, _NG)


def _stage1(ri, spc, aw, ab):
    full = lambda a: pl.BlockSpec(a.shape, lambda i: tuple(0 for _ in a.shape))
    weights = [spc["kw0T"], spc["kb0"], spc["kg0"], spc["kbt0"], spc["kw1T"],
               spc["kb1"], spc["kg1"], spc["kbt1"], spc["kw2T"], spc["kb2"],
               spc["owTe"], spc["owTo"], spc["ob"], spc["lng"], spc["lnb"],
               aw, ab]
    rows = ri.shape[0] // _K
    nb = rows // _P1
    nbat = rows // _N
    per_batch = _N // _P1
    return pl.pallas_call(
        _stage1_body,
        grid=(nb,),
        in_specs=[pl.BlockSpec((_P1 * _K, 8), lambda i: (i, 0))]
                 + [full(a) for a in weights],
        out_specs=[pl.BlockSpec((_P1, _OUT), lambda i: (i, 0)),
                   pl.BlockSpec((1, 1, _NG), lambda i: (i // per_batch, 0, 0))],
        out_shape=[jax.ShapeDtypeStruct((rows, _OUT), jnp.float32),
                   jax.ShapeDtypeStruct((nbat, 1, _NG), jnp.float32)],
        scratch_shapes=[pltpu.VMEM((1, _NG), jnp.float32)],
    )(ri, *weights)


_P2 = 256
_G = 8


def _stage2_body(ri_ref, fn_ref, glob1_ref, kw0T, kb0, kg0, kbt0, kw1T,
                 kb1, kg1, kbt1, kw2T, kb2, w2_ref, ob, lng, lnb, awT, ab,
                 f_ref, glob_ref, acc, c_scr):
    i = pl.program_id(0)
    per_batch = _N // _P2
    b = i // per_batch

    @pl.when(i % per_batch == 0)
    def _():
        acc[...] = jnp.full((1, _NG), -jnp.inf, jnp.float32)

    p = _P2
    w = dict(kw0T=kw0T[...], kb0=kb0[...], kg0=kg0[...], kbt0=kbt0[...],
             kw1T=kw1T[...], kb1=kb1[...], kg1=kg1[...], kbt1=kbt1[...],
             kw2T=kw2T[...], kb2=kb2[...])
    kern = _mlp_kern(ri_ref[...], w)
    kernT = kern.T

    globrow = glob1_ref[pl.ds(b, 1), 0, :]
    fne = jnp.concatenate(
        [fn_ref[...], jnp.broadcast_to(globrow, (p * _K, _NG))], axis=1)

    rows = lax.broadcasted_iota(jnp.int32, (_RANK * _G, _G * _K), 0)
    cols = lax.broadcasted_iota(jnp.int32, (_RANK * _G, _G * _K), 1)
    bd_mask = (rows % _G) == (cols // _K)

    for gi in range(p // _G):
        kt = kernT[:, gi * _G * _K:(gi + 1) * _G * _K]
        tbig = jnp.broadcast_to(kt[:, None, :], (_RANK, _G, _G * _K))
        tbig = tbig.reshape(_RANK * _G, _G * _K)
        a_bd = jnp.where(bd_mask, tbig, 0.0)
        fc = fne[gi * _G * _K:(gi + 1) * _G * _K, :]
        cchunk = jnp.dot(a_bd, fc, preferred_element_type=jnp.float32)
        c_scr[:, pl.ds(gi * _G, _G), :] = cchunk.reshape(_RANK, _G, 160)

    out = ob[...] * jnp.ones((p, _OUT), jnp.float32)
    for r in range(_RANK):
        out = out + jnp.dot(c_scr[r], w2_ref[r],
                            preferred_element_type=jnp.float32)
    out = _ln_mx(out, lng[...], lnb[...], _OUT)
    f_ref[...] = out
    tran = jnp.dot(out, awT[...], preferred_element_type=jnp.float32) + ab[...]
    acc[...] = jnp.maximum(acc[...], jnp.max(tran, axis=0, keepdims=True))
    glob_ref[...] = acc[...].reshape(1, 1, _NG)


def _stage2(ri, fn, glob1, spc, aw, ab):
    full = lambda a: pl.BlockSpec(a.shape, lambda i: tuple(0 for _ in a.shape))
    weights = [spc["kw0T"], spc["kb0"], spc["kg0"], spc["kbt0"], spc["kw1T"],
               spc["kb1"], spc["kg1"], spc["kbt1"], spc["kw2T"], spc["kb2"],
               spc["w2"], spc["ob"], spc["lng"], spc["lnb"], aw, ab]
    rows = ri.shape[0] // _K
    nb = rows // _P2
    nbat = rows // _N
    per_batch = _N // _P2
    return pl.pallas_call(
        _stage2_body,
        grid=(nb,),
        in_specs=[pl.BlockSpec((_P2 * _K, 8), lambda i: (i, 0)),
                  pl.BlockSpec((_P2 * _K, _OUT), lambda i: (i, 0)),
                  full(glob1)]
                 + [full(a) for a in weights],
        out_specs=[pl.BlockSpec((_P2, _OUT), lambda i: (i, 0)),
                   pl.BlockSpec((1, 1, _NG), lambda i: (i // per_batch, 0, 0))],
        out_shape=[jax.ShapeDtypeStruct((rows, _OUT), jnp.float32),
                   jax.ShapeDtypeStruct((nbat, 1, _NG), jnp.float32)],
        scratch_shapes=[pltpu.VMEM((1, _NG), jnp.float32),
                        pltpu.VMEM((_RANK, _P2, 160), jnp.float32)],
    )(ri, fn, glob1, *weights)


_PC = 256


def _concat_body(f_ref, glob_ref, out_ref):
    gl = glob_ref[...].reshape(1, _NG)
    out_ref[...] = jnp.concatenate(
        [f_ref[...], jnp.broadcast_to(gl, (_PC, _NG))], axis=1)


def _concat(f2, glob2):
    rows = f2.shape[0]
    nb = rows // _PC
    per_batch = _N // _PC
    return pl.pallas_call(
        _concat_body,
        grid=(nb,),
        in_specs=[pl.BlockSpec((_PC, _OUT), lambda i: (i, 0)),
                  pl.BlockSpec((1, 1, _NG), lambda i: (i // per_batch, 0, 0))],
        out_specs=pl.BlockSpec((_PC, _OUT + _NG), lambda i: (i, 0)),
        out_shape=jax.ShapeDtypeStruct((rows, _OUT + _NG), jnp.float32),
    )(f2, glob2)


def _prep_spc0(p):
    owT = p["ow"].T
    return dict(
        kw0T=jnp.pad(p["kw0"].T, ((0, 2), (0, 0))), kb0=p["kb0"],
        kg0=p["kg0"], kbt0=p["kbt0"],
        kw1T=p["kw1"].T, kb1=p["kb1"], kg1=p["kg1"], kbt1=p["kbt1"],
        kw2T=p["kw2"].T, kb2=p["kb2"],
        owTe=owT[0::2], owTo=owT[1::2],
        ob=p["ob"], lng=p["lng"], lnb=p["lnb"])


def _prep_spc1(p):
    w2 = p["ow"].reshape(_OUT, _RANK, 160).transpose(1, 2, 0)
    return dict(
        kw0T=jnp.pad(p["kw0"].T, ((0, 2), (0, 0))), kb0=p["kb0"],
        kg0=p["kg0"], kbt0=p["kbt0"],
        kw1T=p["kw1"].T, kb1=p["kb1"], kg1=p["kg1"], kbt1=p["kbt1"],
        kw2T=p["kw2"].T, kb2=p["kb2"],
        w2=w2, ob=p["ob"], lng=p["lng"], lnb=p["lnb"])


def _half_pipeline(pcf, pcnf, dist2, spc0, spc1, aw0, ab0, aw1, ab1):
    rows = pcf.shape[0]
    gidx = _topk(dist2)
    geom8 = jnp.concatenate(
        [pcf, pcnf, jnp.zeros((rows, 2), jnp.float32)], axis=1).reshape(-1)
    ri = _sc_geom(geom8, gidx.reshape(-1)).reshape(rows * _K, 8)
    f1, glob1 = _stage1(ri, spc0, aw0, ab0)
    fn = _sc_gather(f1, gidx.reshape(rows * _K // _CH, _CH), _OUT)
    f2, glob2 = _stage2(ri, fn, glob1, spc1, aw1, ab1)
    return _concat(f2, glob2)


def kernel(pc, pc_normal, dist, params):
    b, n, _ = pc.shape
    pcf = pc.reshape(_BN, 3)
    pcnf = pc_normal.reshape(_BN, 3)
    dist2 = dist.reshape(_BN, _N)

    spc0 = _prep_spc0(params["spc0"])
    spc1 = _prep_spc1(params["spc1"])
    aw0, ab0 = params["aggr0_w"].T, params["aggr0_b"]
    aw1, ab1 = params["aggr1_w"].T, params["aggr1_b"]

    h = _BN // 2
    halves = [
        _half_pipeline(pcf[s], pcnf[s], dist2[s],
                       spc0, spc1, aw0, ab0, aw1, ab1)
        for s in (slice(0, h), slice(h, _BN))
    ]
    out = jnp.concatenate(halves, axis=0)
    return out.reshape(b, n, _OUT + _NG)


# speedup vs baseline: 1.0105x; 1.0105x over previous
import functools

import jax
import jax.numpy as jnp
from jax import lax
from jax.experimental import pallas as pl
from jax.experimental.pallas import tpu as pltpu
from jax.experimental.pallas import tpu_sc as plsc

_B, _N, _K = 4, 2048, 16
_BN = _B * _N
_RANK = 32
_OUT = 128
_NG = 32


_TOPK_ROWS = 256


def _topk_body(dist_ref, idx_ref):
    x = dist_ref[...]
    r = x.shape[0]
    colf = lax.broadcasted_iota(jnp.int32, (r, _N), 1).astype(jnp.float32)
    lane = lax.broadcasted_iota(jnp.int32, (r, _K), 1)
    acc = jnp.zeros((r, _K), jnp.int32)
    for j in range(_K):
        m = jnp.min(x, axis=1, keepdims=True)
        i = jnp.min(jnp.where(x == m, colf, jnp.float32(_N)),
                    axis=1)
        acc = jnp.where(lane == j, i[:, None].astype(jnp.int32), acc)
        x = jnp.where(colf == i[:, None], jnp.inf, x)
    base = pl.program_id(0) * _TOPK_ROWS // _N * _N
    idx_ref[...] = acc + base


def _topk(dist2):
    return pl.pallas_call(
        _topk_body,
        grid=(_BN // _TOPK_ROWS,),
        in_specs=[pl.BlockSpec((_TOPK_ROWS, _N), lambda i: (i, 0))],
        out_specs=pl.BlockSpec((_TOPK_ROWS, _K), lambda i: (i, 0)),
        out_shape=jax.ShapeDtypeStruct((_BN, _K), jnp.int32),
    )(dist2)


_NW = 32
_CH = 128
_GW = 128


def _sc_sqrt(x):
    xi = lax.bitcast_convert_type(x, jnp.int32)
    s = lax.bitcast_convert_type((xi >> 1) + 0x1FBD1DF6, jnp.float32)
    for _ in range(3):
        s = 0.5 * (s + x / s)
    return s


def _sc_geom(table_flat, gidx_flat):
    m = gidx_flat.shape[0]
    per_w = m // _NW
    npts = per_w // _K
    tlen = table_flat.shape[0]
    mesh = plsc.VectorSubcoreMesh(core_axis_name="c", subcore_axis_name="s")

    @functools.partial(
        pl.kernel,
        mesh=mesh,
        out_type=jax.ShapeDtypeStruct((m * 8,), jnp.float32),
        scratch_types=[
            pltpu.VMEM((tlen,), jnp.float32),
            pltpu.VMEM((per_w,), jnp.int32),
            pltpu.VMEM((per_w * 8,), jnp.float32),
        ],
        compiler_params=pltpu.CompilerParams(needs_layout_passes=False),
    )
    def k(table_hbm, idx_hbm, out_hbm, table_v, idx_v, obuf):
        wid = lax.axis_index("s") * 2 + lax.axis_index("c")
        pltpu.sync_copy(table_hbm, table_v)
        pltpu.sync_copy(idx_hbm.at[pl.ds(wid * per_w, per_w)], idx_v)
        lanes = lax.iota(jnp.int32, 16)

        def body(p, carry):
            idx16 = idx_v[pl.ds(p * _K, _K)]
            pg = wid * npts + p

            def nbr(c):
                return plsc.load_gather(table_v, [idx16 * 8 + c])

            def cen(c):
                return plsc.load_gather(
                    table_v, [jnp.full((16,), pg * 8 + c, jnp.int32)])

            px, py, pz = nbr(0), nbr(1), nbr(2)
            nx, ny, nz = nbr(3), nbr(4), nbr(5)
            cx, cy, cz = cen(0), cen(1), cen(2)
            wx, wy, wz = cen(3), cen(4), cen(5)
            mx = jnp.broadcast_to(jnp.sum(px) * (1.0 / _K), (16,))
            my = jnp.broadcast_to(jnp.sum(py) * (1.0 / _K), (16,))
            mz = jnp.broadcast_to(jnp.sum(pz) * (1.0 / _K), (16,))
            l1x, l1y, l1z = mx - px, my - py, mz - pz
            l2x, l2y, l2z = px - cx, py - cy, pz - cz
            l3x, l3y, l3z = cx - mx, cy - my, cz - mz
            l1n = _sc_sqrt(l1x * l1x + l1y * l1y + l1z * l1z)
            l2n = _sc_sqrt(l2x * l2x + l2y * l2y + l2z * l2z)
            l3n = _sc_sqrt(l3x * l3x + l3y * l3y + l3z * l3z)
            t1 = (l1x * l2x + l1y * l2y + l1z * l2z) / (l1n * l2n + 1e-7)
            t2 = (l2x * l3x + l2y * l3y + l2z * l3z) / (l2n * l3n + 1e-7)
            t3 = (l3x * l1x + l3y * l1y + l3z * l1z) / (l3n * l1n + 1e-7)
            cosv = nx * wx + ny * wy + nz * wz
            zero = jnp.zeros((16,), jnp.float32)
            obase = p * (_K * 8)
            for c, val in enumerate([l1n, l2n, l3n, t1, t2, t3, cosv, zero]):
                plsc.store_scatter(obuf, [obase + lanes * 8 + c], val)
            return carry

        lax.fori_loop(0, npts, body, 0)
        pltpu.sync_copy(obuf, out_hbm.at[pl.ds(wid * per_w * 8, per_w * 8)])

    return k(table_flat, gidx_flat)


def _sc_gather(table, gidx2, d):
    m = gidx2.shape[0] * _CH
    per_w = m // _NW
    nch = per_w // _CH
    mesh = plsc.VectorSubcoreMesh(core_axis_name="c", subcore_axis_name="s")

    @functools.partial(
        pl.kernel,
        mesh=mesh,
        out_type=jax.ShapeDtypeStruct((m, d), jnp.float32),
        scratch_types=[
            pltpu.VMEM((nch, _CH), jnp.int32),
            pltpu.VMEM((2, _CH, d), jnp.float32),
            pltpu.SemaphoreType.DMA,
        ],
    )
    def k(table_hbm, idx_hbm, out_hbm, idx_v, buf, sem):
        wid = lax.axis_index("s") * 2 + lax.axis_index("c")
        base = wid * per_w
        pltpu.sync_copy(idx_hbm.at[pl.ds(wid * nch, nch)], idx_v)
        pltpu.async_copy(table_hbm.at[idx_v.at[0]], buf.at[0], sem)

        def body(j, _):
            pltpu.make_async_copy(
                table_hbm.at[idx_v.at[0]], buf.at[0], sem).wait()

            @pl.when(j + 1 < nch)
            def _fire():
                pltpu.async_copy(
                    table_hbm.at[idx_v.at[j + 1]],
                    buf.at[lax.rem(j + 1, 2)], sem)

            pltpu.sync_copy(buf.at[lax.rem(j, 2)],
                            out_hbm.at[pl.ds(base + j * _CH, _CH)])
            return _

        lax.fori_loop(0, nch, body, 0)

    return k(table, gidx2)


def _ln_mx(x, g, b, c):
    mu = jnp.mean(x, axis=-1, keepdims=True)
    xc = x - mu
    v = jnp.mean(xc * xc, axis=-1, keepdims=True)
    return xc / jnp.sqrt(v + 1e-5) * g + b


def _mlp_kern(ri, w):
    h = jnp.dot(ri, w["kw0T"], preferred_element_type=jnp.float32) + w["kb0"]
    h = jnp.maximum(_ln_mx(h, w["kg0"], w["kbt0"], 32), 0.0)
    h = jnp.dot(h, w["kw1T"], preferred_element_type=jnp.float32) + w["kb1"]
    h = jnp.maximum(_ln_mx(h, w["kg1"], w["kbt1"], 32), 0.0)
    return jnp.dot(h, w["kw2T"], preferred_element_type=jnp.float32) + w["kb2"]


_P1 = 256


def _stage1_body(ri_ref, kw0T, kb0, kg0, kbt0, kw1T, kb1, kg1,
                 kbt1, kw2T, kb2, owTe, owTo, ob, lng, lnb, awT, ab,
                 f_ref, glob_ref, acc):
    i = pl.program_id(0)
    per_batch = pl.num_programs(0) // _B

    @pl.when(i % per_batch == 0)
    def _():
        acc[...] = jnp.full((1, _NG), -jnp.inf, jnp.float32)

    p = _P1
    ri = ri_ref[...]
    l2n = ri[:, 1:2].reshape(p, _K, 1)
    cosv = ri[:, 6:7].reshape(p, _K, 1)
    w = dict(kw0T=kw0T[...], kb0=kb0[...], kg0=kg0[...], kbt0=kbt0[...],
             kw1T=kw1T[...], kb1=kb1[...], kg1=kg1[...], kbt1=kbt1[...],
             kw2T=kw2T[...], kb2=kb2[...])
    kern = _mlp_kern(ri, w).reshape(p, _K, _RANK)
    a0 = jnp.sum(kern * l2n, axis=1)
    a1 = jnp.sum(kern * cosv, axis=1)
    out = (jnp.dot(a0, owTe[...], preferred_element_type=jnp.float32)
           + jnp.dot(a1, owTo[...], preferred_element_type=jnp.float32)
           + ob[...])
    out = _ln_mx(out, lng[...], lnb[...], _OUT)
    f_ref[...] = out
    tran = jnp.dot(out, awT[...], preferred_element_type=jnp.float32) + ab[...]
    acc[...] = jnp.maximum(acc[...], jnp.max(tran, axis=0, keepdims=True))
    glob_ref[...] = acc[...].reshape(1, 1, _NG)


def _stage1(ri, spc, aw, ab):
    full = lambda a: pl.BlockSpec(a.shape, lambda i: tuple(0 for _ in a.shape))
    weights = [spc["kw0T"], spc["kb0"], spc["kg0"], spc["kbt0"], spc["kw1T"],
               spc["kb1"], spc["kg1"], spc["kbt1"], spc["kw2T"], spc["kb2"],
               spc["owTe"], spc["owTo"], spc["ob"], spc["lng"], spc["lnb"],
               aw, ab]
    nb = _BN // _P1
    per_batch = nb // _B
    return pl.pallas_call(
        _stage1_body,
        grid=(nb,),
        in_specs=[pl.BlockSpec((_P1 * _K, 8), lambda i: (i, 0))]
                 + [full(a) for a in weights],
        out_specs=[pl.BlockSpec((_P1, _OUT), lambda i: (i, 0)),
                   pl.BlockSpec((1, 1, _NG), lambda i: (i // per_batch, 0, 0))],
        out_shape=[jax.ShapeDtypeStruct((_BN, _OUT), jnp.float32),
                   jax.ShapeDtypeStruct((_B, 1, _NG), jnp.float32)],
        scratch_shapes=[pltpu.VMEM((1, _NG), jnp.float32)],
    )(ri, *weights)


_P2 = 256
_G = 8


def _stage2_body(ri_ref, fn_ref, glob1_ref, kw0T, kb0, kg0, kbt0, kw1T,
                 kb1, kg1, kbt1, kw2T, kb2, w2_ref, ob, lng, lnb, awT, ab,
                 f_ref, glob_ref, acc, c_scr):
    i = pl.program_id(0)
    per_batch = pl.num_programs(0) // _B
    b = i // per_batch

    @pl.when(i % per_batch == 0)
    def _():
        acc[...] = jnp.full((1, _NG), -jnp.inf, jnp.float32)

    p = _P2
    w = dict(kw0T=kw0T[...], kb0=kb0[...], kg0=kg0[...], kbt0=kbt0[...],
             kw1T=kw1T[...], kb1=kb1[...], kg1=kg1[...], kbt1=kbt1[...],
             kw2T=kw2T[...], kb2=kb2[...])
    kern = _mlp_kern(ri_ref[...], w)
    kernT = kern.T

    globrow = glob1_ref[pl.ds(b, 1), 0, :]
    fne = jnp.concatenate(
        [fn_ref[...], jnp.broadcast_to(globrow, (p * _K, _NG))], axis=1)

    rows = lax.broadcasted_iota(jnp.int32, (_RANK * _G, _G * _K), 0)
    cols = lax.broadcasted_iota(jnp.int32, (_RANK * _G, _G * _K), 1)
    bd_mask = (rows % _G) == (cols // _K)

    for gi in range(p // _G):
        kt = kernT[:, gi * _G * _K:(gi + 1) * _G * _K]
        tbig = jnp.broadcast_to(kt[:, None, :], (_RANK, _G, _G * _K))
        tbig = tbig.reshape(_RANK * _G, _G * _K)
        a_bd = jnp.where(bd_mask, tbig, 0.0)
        fc = fne[gi * _G * _K:(gi + 1) * _G * _K, :]
        cchunk = jnp.dot(a_bd, fc, preferred_element_type=jnp.float32)
        c_scr[:, pl.ds(gi * _G, _G), :] = cchunk.reshape(_RANK, _G, 160)

    out = ob[...] * jnp.ones((p, _OUT), jnp.float32)
    for r in range(_RANK):
        out = out + jnp.dot(c_scr[r], w2_ref[r],
                            preferred_element_type=jnp.float32)
    out = _ln_mx(out, lng[...], lnb[...], _OUT)
    f_ref[...] = out
    tran = jnp.dot(out, awT[...], preferred_element_type=jnp.float32) + ab[...]
    acc[...] = jnp.maximum(acc[...], jnp.max(tran, axis=0, keepdims=True))
    glob_ref[...] = acc[...].reshape(1, 1, _NG)


def _stage2(ri, fn, glob1, spc, aw, ab):
    full = lambda a: pl.BlockSpec(a.shape, lambda i: tuple(0 for _ in a.shape))
    weights = [spc["kw0T"], spc["kb0"], spc["kg0"], spc["kbt0"], spc["kw1T"],
               spc["kb1"], spc["kg1"], spc["kbt1"], spc["kw2T"], spc["kb2"],
               spc["w2"], spc["ob"], spc["lng"], spc["lnb"], aw, ab]
    nb = _BN // _P2
    per_batch = nb // _B
    return pl.pallas_call(
        _stage2_body,
        grid=(nb,),
        in_specs=[pl.BlockSpec((_P2 * _K, 8), lambda i: (i, 0)),
                  pl.BlockSpec((_P2 * _K, _OUT), lambda i: (i, 0)),
                  full(glob1)]
                 + [full(a) for a in weights],
        out_specs=[pl.BlockSpec((_P2, _OUT), lambda i: (i, 0)),
                   pl.BlockSpec((1, 1, _NG), lambda i: (i // per_batch, 0, 0))],
        out_shape=[jax.ShapeDtypeStruct((_BN, _OUT), jnp.float32),
                   jax.ShapeDtypeStruct((_B, 1, _NG), jnp.float32)],
        scratch_shapes=[pltpu.VMEM((1, _NG), jnp.float32),
                        pltpu.VMEM((_RANK, _P2, 160), jnp.float32)],
    )(ri, fn, glob1, *weights)


_PC = 256


def _concat_body(f_ref, glob_ref, out_ref):
    gl = glob_ref[...].reshape(1, _NG)
    out_ref[...] = jnp.concatenate(
        [f_ref[...], jnp.broadcast_to(gl, (_PC, _NG))], axis=1)


def _concat(f2, glob2):
    nb = _BN // _PC
    per_batch = nb // _B
    return pl.pallas_call(
        _concat_body,
        grid=(nb,),
        in_specs=[pl.BlockSpec((_PC, _OUT), lambda i: (i, 0)),
                  pl.BlockSpec((1, 1, _NG), lambda i: (i // per_batch, 0, 0))],
        out_specs=pl.BlockSpec((_PC, _OUT + _NG), lambda i: (i, 0)),
        out_shape=jax.ShapeDtypeStruct((_BN, _OUT + _NG), jnp.float32),
    )(f2, glob2)


def _prep_spc0(p):
    owT = p["ow"].T
    return dict(
        kw0T=jnp.pad(p["kw0"].T, ((0, 2), (0, 0))), kb0=p["kb0"],
        kg0=p["kg0"], kbt0=p["kbt0"],
        kw1T=p["kw1"].T, kb1=p["kb1"], kg1=p["kg1"], kbt1=p["kbt1"],
        kw2T=p["kw2"].T, kb2=p["kb2"],
        owTe=owT[0::2], owTo=owT[1::2],
        ob=p["ob"], lng=p["lng"], lnb=p["lnb"])


def _prep_spc1(p):
    w2 = p["ow"].reshape(_OUT, _RANK, 160).transpose(1, 2, 0)
    return dict(
        kw0T=jnp.pad(p["kw0"].T, ((0, 2), (0, 0))), kb0=p["kb0"],
        kg0=p["kg0"], kbt0=p["kbt0"],
        kw1T=p["kw1"].T, kb1=p["kb1"], kg1=p["kg1"], kbt1=p["kbt1"],
        kw2T=p["kw2"].T, kb2=p["kb2"],
        w2=w2, ob=p["ob"], lng=p["lng"], lnb=p["lnb"])


def kernel(pc, pc_normal, dist, params):
    b, n, _ = pc.shape
    pcf = pc.reshape(_BN, 3)
    pcnf = pc_normal.reshape(_BN, 3)
    dist2 = dist.reshape(_BN, _N)

    gidx = _topk(dist2)
    gidx2 = gidx.reshape(_BN * _K // _CH, _CH)

    geom8 = jnp.concatenate(
        [pcf, pcnf, jnp.zeros((_BN, 2), jnp.float32)], axis=1).reshape(-1)
    ri = _sc_geom(geom8, gidx.reshape(-1)).reshape(_BN * _K, 8)

    spc0 = _prep_spc0(params["spc0"])
    f1, glob1 = _stage1(ri, spc0, params["aggr0_w"].T, params["aggr0_b"])

    fn = _sc_gather(f1, gidx2, _OUT)

    spc1 = _prep_spc1(params["spc1"])
    f2, glob2 = _stage2(ri, fn, glob1, spc1, params["aggr1_w"].T,
                        params["aggr1_b"])

    out = _concat(f2, glob2)
    return out.reshape(b, n, _OUT + _NG)
